# Initial kernel scaffold; baseline (speedup 1.0000x reference)
import jax
import jax.numpy as jnp
from jax.experimental import pallas as pl


def kernel(src, emb_table, W):
    raise NotImplementedError("write your pallas kernel here")


# speedup vs baseline: 2.0581x; 2.0581x over previous
import functools

import jax
import jax.numpy as jnp
from jax import lax
from jax.experimental import pallas as pl
from jax.experimental.pallas import tpu as pltpu
from jax.experimental.pallas import tpu_sc as plsc

NUM = 100000
RANK = 32
OUT_DIM = 128
BATCH = 4096
FIELDS = 26
BF = BATCH * FIELDS

NC = 2
NS = 16
NW = NC * NS
BPW = BF // NW

_sc_mesh = plsc.VectorSubcoreMesh(core_axis_name="c", subcore_axis_name="s")


@functools.partial(
    pl.kernel,
    mesh=_sc_mesh,
    out_type=jax.ShapeDtypeStruct((BF, RANK), jnp.float32),
    scratch_types=[
        pltpu.VMEM((BPW,), jnp.int32),
        pltpu.VMEM((BPW, RANK), jnp.float32),
        pltpu.SemaphoreType.DMA,
    ],
    compiler_params=pltpu.CompilerParams(use_tc_tiling_on_sc=False),
)
def _sc_gather(table_hbm, idx_hbm, out_hbm, idx_v, rows_v, sem):
    wid = lax.axis_index("s") * NC + lax.axis_index("c")
    base = wid * BPW
    pltpu.sync_copy(idx_hbm.at[pl.ds(base, BPW)], idx_v)
    pltpu.async_copy(table_hbm.at[idx_v], rows_v, sem).wait()
    pltpu.sync_copy(rows_v, out_hbm.at[pl.ds(base, BPW)])


_MM_BLK = 2048


def _mm_body(x_ref, w_ref, o_ref):
    o_ref[...] = lax.dot_general(
        x_ref[...], w_ref[...],
        (((1,), (1,)), ((), ())),
        preferred_element_type=jnp.float32,
    )


def _tc_matmul(x, w):
    n = x.shape[0]
    grid = (n // _MM_BLK,)
    return pl.pallas_call(
        _mm_body,
        grid=grid,
        in_specs=[
            pl.BlockSpec((_MM_BLK, RANK), lambda i: (i, 0)),
            pl.BlockSpec((OUT_DIM, RANK), lambda i: (0, 0)),
        ],
        out_specs=pl.BlockSpec((_MM_BLK, OUT_DIM), lambda i: (i, 0)),
        out_shape=jax.ShapeDtypeStruct((n, OUT_DIM), jnp.float32),
    )(x, w)


def kernel(src, emb_table, W):
    b, f = src.shape
    idx = src.reshape(-1).astype(jnp.int32)
    gathered = _sc_gather(emb_table, idx)
    out = _tc_matmul(gathered, W)
    return out.reshape(b, f, OUT_DIM)


# speedup vs baseline: 2.4287x; 1.1801x over previous
import functools

import jax
import jax.numpy as jnp
from jax import lax
from jax.experimental import pallas as pl
from jax.experimental.pallas import tpu as pltpu
from jax.experimental.pallas import tpu_sc as plsc

NUM = 100000
RANK = 32
OUT_DIM = 128
BATCH = 4096
FIELDS = 26
BF = BATCH * FIELDS

NC = 2
NS = 16
NW = NC * NS
BPW = BF // NW
CH = 416
NCHUNK = BPW // CH

_sc_mesh = plsc.VectorSubcoreMesh(core_axis_name="c", subcore_axis_name="s")


@functools.partial(
    pl.kernel,
    mesh=_sc_mesh,
    out_type=jax.ShapeDtypeStruct((BF, OUT_DIM), jnp.float32),
    scratch_types=[
        pltpu.VMEM((BPW,), jnp.int32),
        pltpu.VMEM((2, CH, OUT_DIM), jnp.float32),
        pltpu.SemaphoreType.DMA,
        pltpu.SemaphoreType.DMA,
        pltpu.SemaphoreType.DMA,
        pltpu.SemaphoreType.DMA,
    ],
)
def _sc_gather(p_hbm, idx_hbm, out_hbm, idx_v, rows_v, g0, g1, s0, s1):
    wid = lax.axis_index("s") * NC + lax.axis_index("c")
    base = wid * BPW
    pltpu.sync_copy(idx_hbm.at[pl.ds(base, BPW)], idx_v)

    gsem = (g0, g1)
    ssem = (s0, s1)

    def start_gather(j):
        b = j & 1
        return pltpu.async_copy(
            p_hbm.at[idx_v.at[pl.ds(j * CH, CH)]], rows_v.at[b], gsem[b])

    def start_store(j):
        b = j & 1
        return pltpu.async_copy(
            rows_v.at[b], out_hbm.at[pl.ds(base + j * CH, CH)], ssem[b])

    gath = [None, None]
    stor = [None, None]
    gath[0] = start_gather(0)
    for j in range(1, NCHUNK):
        b = j & 1
        pb = (j - 1) & 1
        if stor[b] is not None:
            stor[b].wait()
        gath[b] = start_gather(j)
        gath[pb].wait()
        stor[pb] = start_store(j - 1)
    lb = (NCHUNK - 1) & 1
    gath[lb].wait()
    stor[lb] = start_store(NCHUNK - 1)
    stor[(NCHUNK - 2) & 1].wait()
    stor[lb].wait()


_P_BLK = 5000


def _proj_body(t_ref, w_ref, p_ref):
    p_ref[...] = lax.dot_general(
        t_ref[...], w_ref[...],
        (((1,), (1,)), ((), ())),
        preferred_element_type=jnp.float32,
    )


def _tc_project(table, w):
    return pl.pallas_call(
        _proj_body,
        grid=(NUM // _P_BLK,),
        in_specs=[
            pl.BlockSpec((_P_BLK, RANK), lambda i: (i, 0)),
            pl.BlockSpec((OUT_DIM, RANK), lambda i: (0, 0)),
        ],
        out_specs=pl.BlockSpec((_P_BLK, OUT_DIM), lambda i: (i, 0)),
        out_shape=jax.ShapeDtypeStruct((NUM, OUT_DIM), jnp.float32),
    )(table, w)


def kernel(src, emb_table, W):
    b, f = src.shape
    idx = src.reshape(-1).astype(jnp.int32)
    proj = _tc_project(emb_table, W)
    out = _sc_gather(proj, idx)
    return out.reshape(b, f, OUT_DIM)


# speedup vs baseline: 2.9324x; 1.2074x over previous
import functools

import jax
import jax.numpy as jnp
from jax import lax
from jax.experimental import pallas as pl
from jax.experimental.pallas import tpu as pltpu
from jax.experimental.pallas import tpu_sc as plsc

NUM = 100000
RANK = 32
OUT_DIM = 128
BATCH = 4096
FIELDS = 26
BF = BATCH * FIELDS

NC = 2
NS = 16
NW = NC * NS
ROWS_PW = BATCH // NW
CHB = 16
CH = CHB * FIELDS
NCHUNK = ROWS_PW // CHB

_sc_mesh = plsc.VectorSubcoreMesh(core_axis_name="c", subcore_axis_name="s")


@functools.partial(
    pl.kernel,
    mesh=_sc_mesh,
    out_type=jax.ShapeDtypeStruct((BATCH, FIELDS, OUT_DIM), jnp.float32),
    scratch_types=[
        pltpu.VMEM((ROWS_PW * FIELDS,), jnp.int32),
        pltpu.VMEM((2, CH, OUT_DIM), jnp.float32),
        pltpu.SemaphoreType.DMA,
        pltpu.SemaphoreType.DMA,
        pltpu.SemaphoreType.DMA,
        pltpu.SemaphoreType.DMA,
    ],
)
def _sc_gather(p_hbm, idx_hbm, out_hbm, idx_v, rows_v, g0, g1, s0, s1):
    wid = lax.axis_index("s") * NC + lax.axis_index("c")
    base = wid * ROWS_PW
    pltpu.sync_copy(idx_hbm.at[pl.ds(base * FIELDS, ROWS_PW * FIELDS)], idx_v)

    gsem = (g0, g1)
    ssem = (s0, s1)

    def start_gather(j):
        b = j & 1
        return pltpu.async_copy(
            p_hbm.at[idx_v.at[pl.ds(j * CH, CH)]], rows_v.at[b], gsem[b])

    def start_store(j):
        b = j & 1
        last = None
        for q in range(CHB):
            last = pltpu.async_copy(
                rows_v.at[b].at[pl.ds(q * FIELDS, FIELDS)],
                out_hbm.at[base + j * CHB + q],
                ssem[b])
        return last

    def drain_store(j):
        b = j & 1
        for q in range(CHB):
            pltpu.make_async_copy(
                rows_v.at[b].at[pl.ds(q * FIELDS, FIELDS)],
                out_hbm.at[base + j * CHB + q],
                ssem[b]).wait()

    gath = [None, None]
    stor = [None, None]
    gath[0] = start_gather(0)
    for j in range(1, NCHUNK):
        b = j & 1
        pb = (j - 1) & 1
        if stor[b] is not None:
            drain_store(j - 2)
        gath[b] = start_gather(j)
        gath[pb].wait()
        stor[pb] = start_store(j - 1)
    lb = (NCHUNK - 1) & 1
    gath[lb].wait()
    stor[lb] = start_store(NCHUNK - 1)
    drain_store(NCHUNK - 2)
    drain_store(NCHUNK - 1)


_P_BLK = 1000


def _proj_body(t_ref, w_ref, p_ref):
    for k in range(4):
        y = lax.dot_general(
            t_ref[:, k * RANK:(k + 1) * RANK], w_ref[...],
            (((1,), (1,)), ((), ())),
            preferred_element_type=jnp.float32,
        )
        p_ref[:, k, :] = y


def _tc_project(table_r, w):
    n = table_r.shape[0]
    return pl.pallas_call(
        _proj_body,
        grid=(n // _P_BLK,),
        in_specs=[
            pl.BlockSpec((_P_BLK, 4 * RANK), lambda i: (i, 0)),
            pl.BlockSpec((OUT_DIM, RANK), lambda i: (0, 0)),
        ],
        out_specs=pl.BlockSpec((_P_BLK, 4, OUT_DIM), lambda i: (i, 0, 0)),
        out_shape=jax.ShapeDtypeStruct((n, 4, OUT_DIM), jnp.float32),
    )(table_r, w)


def kernel(src, emb_table, W):
    b, f = src.shape
    idx = src.reshape(-1).astype(jnp.int32)
    table_r = emb_table.reshape(NUM // 4, 4 * RANK)
    proj = _tc_project(table_r, W).reshape(NUM, OUT_DIM)
    return _sc_gather(proj, idx)


# speedup vs baseline: 6.3594x; 2.1687x over previous
import functools

import jax
import jax.numpy as jnp
from jax import lax
from jax.experimental import pallas as pl
from jax.experimental.pallas import tpu as pltpu
from jax.experimental.pallas import tpu_sc as plsc

NUM = 100000
RANK = 32
OUT_DIM = 128
BATCH = 4096
FIELDS = 26
BF = BATCH * FIELDS

NC = 2
NS = 16
NW = NC * NS
BPW = BF // NW
CH = 416
NCHUNK = BPW // CH

_P_COLS = 4096
_P_GRID = 25
_P_ROWS = _P_COLS * _P_GRID

_sc_mesh = plsc.VectorSubcoreMesh(core_axis_name="c", subcore_axis_name="s")


@functools.partial(
    pl.kernel,
    mesh=_sc_mesh,
    out_type=jax.ShapeDtypeStruct((BF, OUT_DIM), jnp.float32),
    scratch_types=[
        pltpu.VMEM((BPW,), jnp.int32),
        pltpu.VMEM((2, CH, OUT_DIM), jnp.float32),
        pltpu.SemaphoreType.DMA,
        pltpu.SemaphoreType.DMA,
        pltpu.SemaphoreType.DMA,
        pltpu.SemaphoreType.DMA,
    ],
)
def _sc_gather(p_hbm, idx_hbm, out_hbm, idx_v, rows_v, g0, g1, s0, s1):
    wid = lax.axis_index("s") * NC + lax.axis_index("c")
    base = wid * BPW
    pltpu.sync_copy(idx_hbm.at[pl.ds(base, BPW)], idx_v)

    gsem = (g0, g1)
    ssem = (s0, s1)

    def start_gather(j):
        b = j & 1
        return pltpu.async_copy(
            p_hbm.at[idx_v.at[pl.ds(j * CH, CH)]], rows_v.at[b], gsem[b])

    def start_store(j):
        b = j & 1
        return pltpu.async_copy(
            rows_v.at[b], out_hbm.at[pl.ds(base + j * CH, CH)], ssem[b])

    gath = [None, None]
    stor = [None, None]
    gath[0] = start_gather(0)
    for j in range(1, NCHUNK):
        b = j & 1
        pb = (j - 1) & 1
        if stor[b] is not None:
            stor[b].wait()
        gath[b] = start_gather(j)
        gath[pb].wait()
        stor[pb] = start_store(j - 1)
    lb = (NCHUNK - 1) & 1
    gath[lb].wait()
    stor[lb] = start_store(NCHUNK - 1)
    stor[(NCHUNK - 2) & 1].wait()
    stor[lb].wait()


def _proj_body(t_ref, w_ref, p_ref):
    p_ref[...] = lax.dot_general(
        t_ref[...], w_ref[...],
        (((0,), (0,)), ((), ())),
        preferred_element_type=jnp.float32,
    )


def _tc_project(table_t, w_t):
    return pl.pallas_call(
        _proj_body,
        grid=(_P_GRID,),
        in_specs=[
            pl.BlockSpec((RANK, _P_COLS), lambda i: (0, i)),
            pl.BlockSpec((RANK, OUT_DIM), lambda i: (0, 0)),
        ],
        out_specs=pl.BlockSpec((_P_COLS, OUT_DIM), lambda i: (i, 0)),
        out_shape=jax.ShapeDtypeStruct((_P_ROWS, OUT_DIM), jnp.float32),
    )(table_t, w_t)


def kernel(src, emb_table, W):
    idx = jnp.transpose(src).reshape(-1).astype(jnp.int32)
    proj = _tc_project(jnp.transpose(emb_table), jnp.transpose(W))
    g = _sc_gather(proj, idx)
    return jnp.transpose(g.reshape(FIELDS, BATCH, OUT_DIM), (1, 0, 2))


# speedup vs baseline: 6.7834x; 1.0667x over previous
import functools

import jax
import jax.numpy as jnp
from jax import lax
from jax.experimental import pallas as pl
from jax.experimental.pallas import tpu as pltpu
from jax.experimental.pallas import tpu_sc as plsc

NUM = 100000
RANK = 32
OUT_DIM = 128
BATCH = 4096
FIELDS = 26
BF = BATCH * FIELDS

NC = 2
NS = 16
NW = NC * NS
BPW = BF // NW
CH = 416
NCHUNK = BPW // CH

_P_COLS = 8192
_P_GRID = 13
_P_ROWS = _P_COLS * _P_GRID

_sc_mesh = plsc.VectorSubcoreMesh(core_axis_name="c", subcore_axis_name="s")


@functools.partial(
    pl.kernel,
    mesh=_sc_mesh,
    out_type=jax.ShapeDtypeStruct((BF, OUT_DIM), jnp.float32),
    scratch_types=[
        pltpu.VMEM((BPW,), jnp.int32),
        pltpu.VMEM((2, CH, OUT_DIM), jnp.float32),
        pltpu.SemaphoreType.DMA,
        pltpu.SemaphoreType.DMA,
        pltpu.SemaphoreType.DMA,
        pltpu.SemaphoreType.DMA,
    ],
)
def _sc_gather(p_hbm, idx_hbm, out_hbm, idx_v, rows_v, g0, g1, s0, s1):
    wid = lax.axis_index("s") * NC + lax.axis_index("c")
    base = wid * BPW
    pltpu.sync_copy(idx_hbm.at[pl.ds(base, BPW)], idx_v)

    gsem = (g0, g1)
    ssem = (s0, s1)

    def start_gather(j):
        b = j & 1
        return pltpu.async_copy(
            p_hbm.at[idx_v.at[pl.ds(j * CH, CH)]], rows_v.at[b], gsem[b])

    def start_store(j):
        b = j & 1
        return pltpu.async_copy(
            rows_v.at[b], out_hbm.at[pl.ds(base + j * CH, CH)], ssem[b])

    gath = [None, None]
    stor = [None, None]
    gath[0] = start_gather(0)
    for j in range(1, NCHUNK):
        b = j & 1
        pb = (j - 1) & 1
        if stor[b] is not None:
            stor[b].wait()
        gath[b] = start_gather(j)
        gath[pb].wait()
        stor[pb] = start_store(j - 1)
    lb = (NCHUNK - 1) & 1
    gath[lb].wait()
    stor[lb] = start_store(NCHUNK - 1)
    stor[(NCHUNK - 2) & 1].wait()
    stor[lb].wait()


def _proj_body(t_ref, w_ref, p_ref):
    p_ref[...] = lax.dot_general(
        t_ref[...], w_ref[...],
        (((0,), (0,)), ((), ())),
        preferred_element_type=jnp.float32,
    )


def _tc_project(table_t, w_t):
    return pl.pallas_call(
        _proj_body,
        grid=(_P_GRID,),
        in_specs=[
            pl.BlockSpec((RANK, _P_COLS), lambda i: (0, i)),
            pl.BlockSpec((RANK, OUT_DIM), lambda i: (0, 0)),
        ],
        out_specs=pl.BlockSpec((_P_COLS, OUT_DIM), lambda i: (i, 0)),
        out_shape=jax.ShapeDtypeStruct((_P_ROWS, OUT_DIM), jnp.float32),
    )(table_t, w_t)


def kernel(src, emb_table, W):
    idx = jnp.transpose(src).reshape(-1).astype(jnp.int32)
    proj = _tc_project(jnp.transpose(emb_table), jnp.transpose(W))
    g = _sc_gather(proj, idx)
    return jnp.transpose(g.reshape(FIELDS, BATCH, OUT_DIM), (1, 0, 2))


# speedup vs baseline: 6.9563x; 1.0255x over previous
import functools

import jax
import jax.numpy as jnp
from jax import lax
from jax.experimental import pallas as pl
from jax.experimental.pallas import tpu as pltpu
from jax.experimental.pallas import tpu_sc as plsc

NUM = 100000
RANK = 32
OUT_DIM = 128
BATCH = 4096
FIELDS = 26
BF = BATCH * FIELDS

NC = 2
NS = 16
NW = NC * NS
BPW = BF // NW
CH = 416
NCHUNK = BPW // CH

_P_COLS = 10240
_P_GRID = 10
_P_ROWS = _P_COLS * _P_GRID

_sc_mesh = plsc.VectorSubcoreMesh(core_axis_name="c", subcore_axis_name="s")


@functools.partial(
    pl.kernel,
    mesh=_sc_mesh,
    out_type=jax.ShapeDtypeStruct((BF, OUT_DIM), jnp.float32),
    scratch_types=[
        pltpu.VMEM((BPW,), jnp.int32),
        pltpu.VMEM((2, CH, OUT_DIM), jnp.float32),
        pltpu.SemaphoreType.DMA,
        pltpu.SemaphoreType.DMA,
        pltpu.SemaphoreType.DMA,
        pltpu.SemaphoreType.DMA,
    ],
)
def _sc_gather(p_hbm, idx_hbm, out_hbm, idx_v, rows_v, g0, g1, s0, s1):
    wid = lax.axis_index("s") * NC + lax.axis_index("c")
    base = wid * BPW
    pltpu.sync_copy(idx_hbm.at[pl.ds(base, BPW)], idx_v)

    gsem = (g0, g1)
    ssem = (s0, s1)

    def start_gather(j):
        b = j & 1
        return pltpu.async_copy(
            p_hbm.at[idx_v.at[pl.ds(j * CH, CH)]], rows_v.at[b], gsem[b])

    def start_store(j):
        b = j & 1
        return pltpu.async_copy(
            rows_v.at[b], out_hbm.at[pl.ds(base + j * CH, CH)], ssem[b])

    gath = [None, None]
    stor = [None, None]
    gath[0] = start_gather(0)
    for j in range(1, NCHUNK):
        b = j & 1
        pb = (j - 1) & 1
        if stor[b] is not None:
            stor[b].wait()
        gath[b] = start_gather(j)
        gath[pb].wait()
        stor[pb] = start_store(j - 1)
    lb = (NCHUNK - 1) & 1
    gath[lb].wait()
    stor[lb] = start_store(NCHUNK - 1)
    stor[(NCHUNK - 2) & 1].wait()
    stor[lb].wait()


def _proj_body(t_ref, w_ref, p_ref):
    p_ref[...] = lax.dot_general(
        t_ref[...], w_ref[...],
        (((0,), (0,)), ((), ())),
        preferred_element_type=jnp.float32,
    )


def _tc_project(table_t, w_t):
    return pl.pallas_call(
        _proj_body,
        grid=(_P_GRID,),
        in_specs=[
            pl.BlockSpec((RANK, _P_COLS), lambda i: (0, i)),
            pl.BlockSpec((RANK, OUT_DIM), lambda i: (0, 0)),
        ],
        out_specs=pl.BlockSpec((_P_COLS, OUT_DIM), lambda i: (i, 0)),
        out_shape=jax.ShapeDtypeStruct((_P_ROWS, OUT_DIM), jnp.float32),
    )(table_t, w_t)


def kernel(src, emb_table, W):
    idx = jnp.transpose(src).reshape(-1).astype(jnp.int32)
    proj = _tc_project(jnp.transpose(emb_table), jnp.transpose(W))
    g = _sc_gather(proj, idx)
    return jnp.transpose(g.reshape(FIELDS, BATCH, OUT_DIM), (1, 0, 2))
